# Initial kernel scaffold; baseline (speedup 1.0000x reference)
import jax
import jax.numpy as jnp
from jax.experimental import pallas as pl


def kernel(x, W, H):
    raise NotImplementedError("write your pallas kernel here")


# speedup vs baseline: 1.1424x; 1.1424x over previous
import functools

import jax
import jax.numpy as jnp
from jax import lax
from jax.experimental import pallas as pl
from jax.experimental.pallas import tpu as pltpu
from jax.experimental.pallas import tpu_sc as plsc

BATCH = 16384
EMBED_K = 128
LANES = 16
NUM_CORES = 2
NUM_SUBCORES = 16
NUM_WORKERS = NUM_CORES * NUM_SUBCORES
ROWS_PER_WORKER = BATCH // NUM_WORKERS
CHUNK = 128
NCHUNK = ROWS_PER_WORKER // CHUNK
KCHUNKS = EMBED_K // LANES
GROUPS = CHUNK // LANES


def _make_kernel():
    mesh = plsc.VectorSubcoreMesh(core_axis_name="c", subcore_axis_name="s")

    @functools.partial(
        pl.kernel,
        mesh=mesh,
        out_type=jax.ShapeDtypeStruct((BATCH,), jnp.float32),
        scratch_types=[
            pltpu.VMEM((NCHUNK, CHUNK), jnp.int32),
            pltpu.VMEM((NCHUNK, CHUNK), jnp.int32),
            pltpu.VMEM((2, CHUNK, EMBED_K), jnp.float32),
            pltpu.VMEM((2, CHUNK, EMBED_K), jnp.float32),
            pltpu.VMEM((ROWS_PER_WORKER,), jnp.float32),
            pltpu.SemaphoreType.DMA,
            pltpu.SemaphoreType.DMA,
            pltpu.SemaphoreType.DMA,
            pltpu.SemaphoreType.DMA,
        ],
    )
    def mf_dot(uidx_hbm, vidx_hbm, w_hbm, h_hbm, out_hbm,
               uidx_v, vidx_v, ubuf, vbuf, outv,
               sem_u0, sem_u1, sem_v0, sem_v1):
        sem_u = (sem_u0, sem_u1)
        sem_v = (sem_v0, sem_v1)
        wid = lax.axis_index("s") * NUM_CORES + lax.axis_index("c")
        base = wid * ROWS_PER_WORKER

        pltpu.sync_copy(uidx_hbm.at[wid], uidx_v)
        pltpu.sync_copy(vidx_hbm.at[wid], vidx_v)

        def start(c):
            slot = c % 2
            cu = pltpu.async_copy(w_hbm.at[uidx_v.at[c]], ubuf.at[slot],
                                  sem_u[slot])
            cv = pltpu.async_copy(h_hbm.at[vidx_v.at[c]], vbuf.at[slot],
                                  sem_v[slot])
            return cu, cv

        iota = lax.iota(jnp.int32, LANES)
        perms = [iota ^ sh for sh in (8, 4, 2, 1)]
        onehot = [iota == l for l in range(LANES)]
        gdn = lax.GatherDimensionNumbers(
            offset_dims=(), collapsed_slice_dims=(0,), start_index_map=(0,))

        def lane_shuffle(x, idx):
            return lax.gather(
                x, idx[:, None], gdn, slice_sizes=(1,),
                mode=lax.GatherScatterMode.PROMISE_IN_BOUNDS)

        def compute(c):
            slot = c % 2
            ub = ubuf.at[slot]
            vb = vbuf.at[slot]

            def group_body(g, carry):
                rbase = g * LANES
                outvec = jnp.zeros((LANES,), jnp.float32)
                for l in range(LANES):
                    r = rbase + l
                    acc = None
                    for i in range(KCHUNKS):
                        p = (ub[r, pl.ds(i * LANES, LANES)]
                             * vb[r, pl.ds(i * LANES, LANES)])
                        acc = p if acc is None else acc + p
                    for perm in perms:
                        acc = acc + lane_shuffle(acc, perm)
                    outvec = jnp.where(onehot[l], acc, outvec)
                outv[pl.ds(c * CHUNK + rbase, LANES)] = outvec
                return carry

            lax.fori_loop(0, GROUPS, group_body, 0)

        pending = start(0)
        for c in range(NCHUNK):
            nxt = start(c + 1) if c + 1 < NCHUNK else None
            pending[0].wait()
            pending[1].wait()
            compute(c)
            pending = nxt

        pltpu.sync_copy(outv, out_hbm.at[pl.ds(base, ROWS_PER_WORKER)])

    return mf_dot


_mf_dot = _make_kernel()


@jax.jit
def kernel(x, W, H):
    uidx = x[:, 0].astype(jnp.int32).reshape(NUM_WORKERS, NCHUNK, CHUNK)
    vidx = x[:, 1].astype(jnp.int32).reshape(NUM_WORKERS, NCHUNK, CHUNK)
    return _mf_dot(uidx, vidx, W, H)
